# baseline (device time: 39887 ns/iter reference)
import os

import jax
import jax.numpy as jnp
from jax import lax
from jax.experimental import pallas as pl
from jax.experimental.pallas import tpu as pltpu

COMPUTE_ONLY = os.environ.get("KERNEL_COMPUTE_ONLY") == "1"

N_DEV = 4
G = 8
SCALE = 0.08838834764831843


def kernel(x, Wq, Wo, K_ext, V_ext):
    B, Sq, D = x.shape
    _, Skv, Hq, Dh = K_ext.shape
    HPG = Hq // G

    x2 = x.reshape(Sq, D)
    K2 = K_ext.reshape(Skv, Hq, Dh)
    V2 = V_ext.reshape(Skv, Hq, Dh)

    def body(x_ref, wq_ref, wo_ref, k_hbm, v_hbm, out_ref,
             comm_ref, kv_buf, send_sems, recv_sems, kv_sems):
        my = lax.axis_index("i")
        p1 = jnp.bitwise_xor(my, 1)
        p2 = jnp.bitwise_xor(my, 3)

        part1 = [p1 if g % 2 == 0 else p2 for g in range(G)]
        part2 = [p2 if g % 2 == 0 else p1 for g in range(G)]

        NBUF = 3

        def kv_fetch(h):
            slot = h % NBUF
            ck = pltpu.make_async_copy(
                k_hbm.at[:, h, :], kv_buf.at[slot, 0], kv_sems.at[slot, 0])
            cv = pltpu.make_async_copy(
                v_hbm.at[:, h, :], kv_buf.at[slot, 1], kv_sems.at[slot, 1])
            ck.start()
            cv.start()
            return ck, cv

        fetches = [kv_fetch(0), kv_fetch(1)]

        barrier_sem = pltpu.get_barrier_semaphore()
        for nbr in (p1, p2):
            pl.semaphore_signal(barrier_sem, inc=1, device_id=(nbr,),
                                device_id_type=pl.DeviceIdType.MESH)
        pl.semaphore_wait(barrier_sem, 2)

        q = jnp.dot(x_ref[...].astype(jnp.bfloat16),
                    wq_ref[...].astype(jnp.bfloat16),
                    preferred_element_type=jnp.float32) * SCALE

        lv = [None] * G
        rdma1 = [None] * G
        rdma2 = [None] * G
        out_parts = []

        def compute_group(g):
            l_cols = []
            for hh in range(HPG):
                h = g * HPG + hh
                slot = h % NBUF
                if h + 2 < Hq:
                    fetches.append(kv_fetch(h + 2))
                cur = fetches[h]
                cur[0].wait()
                cur[1].wait()
                qh = q[:, h * Dh:(h + 1) * Dh].astype(jnp.bfloat16)
                s = lax.dot_general(qh,
                                    kv_buf[slot, 0].astype(jnp.bfloat16),
                                    (((1,), (1,)), ((), ())),
                                    preferred_element_type=jnp.float32)
                p = jnp.exp(s)
                lh = jnp.sum(p, axis=1, keepdims=True)
                oh = jnp.dot(p.astype(jnp.bfloat16),
                             kv_buf[slot, 1].astype(jnp.bfloat16),
                             preferred_element_type=jnp.float32)
                l_cols.append(lh)
                comm_ref[g, 0, hh, :, :] = oh.astype(jnp.bfloat16)
            l_g = jnp.concatenate(l_cols, axis=1)
            comm_ref[g, 0, HPG, :, 0:HPG] = l_g.astype(jnp.bfloat16)
            lv[g] = l_g
            if COMPUTE_ONLY:
                return
            rdma1[g] = pltpu.make_async_remote_copy(
                src_ref=comm_ref.at[g, 0],
                dst_ref=comm_ref.at[g, 1],
                send_sem=send_sems.at[g, 0],
                recv_sem=recv_sems.at[g, 0],
                device_id=(part1[g],),
                device_id_type=pl.DeviceIdType.MESH,
            )
            rdma1[g].start()

        def step1_merge(g):
            rdma1[g].wait()
            l1 = lv[g] + comm_ref[g, 1, HPG, :, 0:HPG].astype(jnp.float32)
            for hh in range(HPG):
                comm_ref[g, 2, hh, :, :] = (comm_ref[g, 0, hh, :, :]
                                            + comm_ref[g, 1, hh, :, :])
            comm_ref[g, 2, HPG, :, 0:HPG] = l1.astype(jnp.bfloat16)
            lv[g] = l1
            rdma2[g] = pltpu.make_async_remote_copy(
                src_ref=comm_ref.at[g, 2],
                dst_ref=comm_ref.at[g, 3],
                send_sem=send_sems.at[g, 1],
                recv_sem=recv_sems.at[g, 1],
                device_id=(part2[g],),
                device_id_type=pl.DeviceIdType.MESH,
            )
            rdma2[g].start()

        def step2_final(g):
            rdma2[g].wait()
            l2 = lv[g] + comm_ref[g, 3, HPG, :, 0:HPG].astype(jnp.float32)
            o_g = jnp.concatenate(
                [(comm_ref[g, 2, hh, :, :].astype(jnp.float32)
                  + comm_ref[g, 3, hh, :, :].astype(jnp.float32))
                 / l2[:, hh:hh + 1]
                 for hh in range(HPG)], axis=1)
            out_parts.append(
                jnp.dot(o_g.astype(jnp.bfloat16),
                        wo_ref[g * HPG * Dh:(g + 1) * HPG * Dh,
                               :].astype(jnp.bfloat16),
                        preferred_element_type=jnp.float32))

        for g in range(G):
            compute_group(g)
            if COMPUTE_ONLY:
                continue
            if g >= 1:
                step1_merge(g - 1)
            if g >= 2:
                step2_final(g - 2)

        if COMPUTE_ONLY:
            for g in range(G):
                o_g = jnp.concatenate(
                    [comm_ref[g, 0, hh, :, :].astype(jnp.float32)
                     / lv[g][:, hh:hh + 1] for hh in range(HPG)], axis=1)
                out_parts.append(
                    jnp.dot(o_g.astype(jnp.bfloat16),
                            wo_ref[g * HPG * Dh:(g + 1) * HPG * Dh,
                                   :].astype(jnp.bfloat16),
                            preferred_element_type=jnp.float32))
        else:
            step1_merge(G - 1)
            for g in range(max(G - 2, 0), G):
                step2_final(g)

        out_val = out_parts[0]
        for part in out_parts[1:]:
            out_val = out_val + part
        out_ref[...] = out_val

    out = pl.pallas_call(
        body,
        out_shape=jax.ShapeDtypeStruct((Sq, D), jnp.float32),
        in_specs=[
            pl.BlockSpec(memory_space=pltpu.VMEM),
            pl.BlockSpec(memory_space=pltpu.VMEM),
            pl.BlockSpec(memory_space=pltpu.VMEM),
            pl.BlockSpec(memory_space=pl.ANY),
            pl.BlockSpec(memory_space=pl.ANY),
        ],
        out_specs=pl.BlockSpec(memory_space=pltpu.VMEM),
        scratch_shapes=[
            pltpu.VMEM((G, 4, Hq // G + 1, Sq, Dh), jnp.bfloat16),
            pltpu.VMEM((3, 2, Skv, Dh), jnp.float32),
            pltpu.SemaphoreType.DMA((G, 2)),
            pltpu.SemaphoreType.DMA((G, 2)),
            pltpu.SemaphoreType.DMA((3, 2)),
        ],
        compiler_params=pltpu.CompilerParams(
            collective_id=0, vmem_limit_bytes=100 * 1024 * 1024),
    )(x2, Wq, Wo, K2, V2)
    return out.reshape(B, Sq, D)


# device time: 34630 ns/iter; 1.1518x vs baseline; 1.1518x over previous
import os

import jax
import jax.numpy as jnp
from jax import lax
from jax.experimental import pallas as pl
from jax.experimental.pallas import tpu as pltpu

COMPUTE_ONLY = os.environ.get("KERNEL_COMPUTE_ONLY") == "1"

N_DEV = 4
G = 4
SCALE = 0.08838834764831843


def kernel(x, Wq, Wo, K_ext, V_ext):
    B, Sq, D = x.shape
    _, Skv, Hq, Dh = K_ext.shape
    HPG = Hq // G

    x2 = x.reshape(Sq, D)
    K2 = K_ext.reshape(Skv, Hq, Dh)
    V2 = V_ext.reshape(Skv, Hq, Dh)

    def body(x_ref, wq_ref, wo_ref, k_hbm, v_hbm, out_ref,
             comm_ref, kv_buf, send_sems, recv_sems, kv_sems):
        my = lax.axis_index("i")
        p1 = jnp.bitwise_xor(my, 1)
        p2 = jnp.bitwise_xor(my, 3)

        part1 = [p1 if g % 2 == 0 else p2 for g in range(G)]
        part2 = [p2 if g % 2 == 0 else p1 for g in range(G)]

        NBUF = 3

        def kv_fetch(h):
            slot = h % NBUF
            ck = pltpu.make_async_copy(
                k_hbm.at[:, h, :], kv_buf.at[slot, 0], kv_sems.at[slot, 0])
            cv = pltpu.make_async_copy(
                v_hbm.at[:, h, :], kv_buf.at[slot, 1], kv_sems.at[slot, 1])
            ck.start()
            cv.start()
            return ck, cv

        fetches = [kv_fetch(0), kv_fetch(1)]

        barrier_sem = pltpu.get_barrier_semaphore()
        for nbr in (p1, p2):
            pl.semaphore_signal(barrier_sem, inc=1, device_id=(nbr,),
                                device_id_type=pl.DeviceIdType.MESH)
        pl.semaphore_wait(barrier_sem, 2)

        q = jnp.dot(x_ref[...].astype(jnp.bfloat16),
                    wq_ref[...].astype(jnp.bfloat16),
                    preferred_element_type=jnp.float32) * SCALE

        lv = [None] * G
        rdma1 = [None] * G
        rdma2 = [None] * G
        out_parts = []

        def compute_group(g):
            l_cols = []
            for hh in range(HPG):
                h = g * HPG + hh
                slot = h % NBUF
                if h + 2 < Hq:
                    fetches.append(kv_fetch(h + 2))
                cur = fetches[h]
                cur[0].wait()
                cur[1].wait()
                qh = q[:, h * Dh:(h + 1) * Dh].astype(jnp.bfloat16)
                s = lax.dot_general(qh,
                                    kv_buf[slot, 0].astype(jnp.bfloat16),
                                    (((1,), (1,)), ((), ())),
                                    preferred_element_type=jnp.float32)
                p = jnp.exp(s)
                lh = jnp.sum(p, axis=1, keepdims=True)
                oh = jnp.dot(p.astype(jnp.bfloat16),
                             kv_buf[slot, 1].astype(jnp.bfloat16),
                             preferred_element_type=jnp.float32)
                l_cols.append(lh)
                comm_ref[g, 0, hh, :, :] = oh.astype(jnp.bfloat16)
            l_g = jnp.concatenate(l_cols, axis=1)
            comm_ref[g, 0, HPG, :, 0:HPG] = l_g.astype(jnp.bfloat16)
            lv[g] = l_g
            if COMPUTE_ONLY:
                return
            rdma1[g] = pltpu.make_async_remote_copy(
                src_ref=comm_ref.at[g, 0],
                dst_ref=comm_ref.at[g, 1],
                send_sem=send_sems.at[g, 0],
                recv_sem=recv_sems.at[g, 0],
                device_id=(part1[g],),
                device_id_type=pl.DeviceIdType.MESH,
            )
            rdma1[g].start()

        def step1_merge(g):
            rdma1[g].wait()
            l1 = lv[g] + comm_ref[g, 1, HPG, :, 0:HPG].astype(jnp.float32)
            for hh in range(HPG):
                comm_ref[g, 2, hh, :, :] = (comm_ref[g, 0, hh, :, :]
                                            + comm_ref[g, 1, hh, :, :])
            comm_ref[g, 2, HPG, :, 0:HPG] = l1.astype(jnp.bfloat16)
            lv[g] = l1
            rdma2[g] = pltpu.make_async_remote_copy(
                src_ref=comm_ref.at[g, 2],
                dst_ref=comm_ref.at[g, 3],
                send_sem=send_sems.at[g, 1],
                recv_sem=recv_sems.at[g, 1],
                device_id=(part2[g],),
                device_id_type=pl.DeviceIdType.MESH,
            )
            rdma2[g].start()

        def step2_final(g):
            rdma2[g].wait()
            l2 = lv[g] + comm_ref[g, 3, HPG, :, 0:HPG].astype(jnp.float32)
            o_g = jnp.concatenate(
                [(comm_ref[g, 2, hh, :, :].astype(jnp.float32)
                  + comm_ref[g, 3, hh, :, :].astype(jnp.float32))
                 / l2[:, hh:hh + 1]
                 for hh in range(HPG)], axis=1)
            out_parts.append(
                jnp.dot(o_g.astype(jnp.bfloat16),
                        wo_ref[g * HPG * Dh:(g + 1) * HPG * Dh,
                               :].astype(jnp.bfloat16),
                        preferred_element_type=jnp.float32))

        for g in range(G):
            compute_group(g)
            if COMPUTE_ONLY:
                continue
            if g >= 1:
                step1_merge(g - 1)
            if g >= 2:
                step2_final(g - 2)

        if COMPUTE_ONLY:
            for g in range(G):
                o_g = jnp.concatenate(
                    [comm_ref[g, 0, hh, :, :].astype(jnp.float32)
                     / lv[g][:, hh:hh + 1] for hh in range(HPG)], axis=1)
                out_parts.append(
                    jnp.dot(o_g.astype(jnp.bfloat16),
                            wo_ref[g * HPG * Dh:(g + 1) * HPG * Dh,
                                   :].astype(jnp.bfloat16),
                            preferred_element_type=jnp.float32))
        else:
            step1_merge(G - 1)
            for g in range(max(G - 2, 0), G):
                step2_final(g)

        out_val = out_parts[0]
        for part in out_parts[1:]:
            out_val = out_val + part
        out_ref[...] = out_val

    out = pl.pallas_call(
        body,
        out_shape=jax.ShapeDtypeStruct((Sq, D), jnp.float32),
        in_specs=[
            pl.BlockSpec(memory_space=pltpu.VMEM),
            pl.BlockSpec(memory_space=pltpu.VMEM),
            pl.BlockSpec(memory_space=pltpu.VMEM),
            pl.BlockSpec(memory_space=pl.ANY),
            pl.BlockSpec(memory_space=pl.ANY),
        ],
        out_specs=pl.BlockSpec(memory_space=pltpu.VMEM),
        scratch_shapes=[
            pltpu.VMEM((G, 4, Hq // G + 1, Sq, Dh), jnp.bfloat16),
            pltpu.VMEM((3, 2, Skv, Dh), jnp.float32),
            pltpu.SemaphoreType.DMA((G, 2)),
            pltpu.SemaphoreType.DMA((G, 2)),
            pltpu.SemaphoreType.DMA((3, 2)),
        ],
        compiler_params=pltpu.CompilerParams(
            collective_id=0, vmem_limit_bytes=100 * 1024 * 1024),
    )(x2, Wq, Wo, K2, V2)
    return out.reshape(B, Sq, D)
